# baseline (speedup 1.0000x reference)
import functools

import jax
import jax.numpy as jnp
from jax import lax
from jax.experimental import pallas as pl
from jax.experimental.pallas import tpu as pltpu
from jax.experimental.pallas import tpu_sc as plsc

N_NODES = 10000
N_EDGES = 320000
NFEAT = 128
NCLASS = 40

NC = 2
NS = 16
NW = NC * NS

CHUNK = 128
NCHUNK = 80
E_PAD = NW * NCHUNK * CHUNK
N_ACC = N_NODES + 16
ROWS_PER_TILE = 640
ROW_CHUNK = 80


def _spmm_body(h_hbm, src_hbm, dst_hbm, out_hbm,
               srcs_v, dsts_v, rows_v, sem_i, sem_g, sem_s, acc_sh):
    c = lax.axis_index("c")
    s = lax.axis_index("s")
    wid = c * NS + s
    r0 = s * ROWS_PER_TILE
    n_row_chunks = jnp.where(s == NS - 1, 5, 8)

    src_idx = pltpu.async_copy(src_hbm.at[wid], srcs_v, sem_i)

    def init_body(k, carry):
        off = r0 + k * ROW_CHUNK
        pltpu.sync_copy(h_hbm.at[pl.ds(off, ROW_CHUNK)],
                        acc_sh.at[pl.ds(off, ROW_CHUNK)])
        return carry
    lax.fori_loop(0, n_row_chunks, init_body, 0)
    src_idx.wait()
    plsc.subcore_barrier()

    def issue_dst(k):
        pltpu.async_copy(dst_hbm.at[wid, k], dsts_v.at[lax.rem(k, 2)], sem_i)

    def wait_dst(k):
        pltpu.make_async_copy(dst_hbm.at[wid, k],
                              dsts_v.at[lax.rem(k, 2)], sem_i).wait()

    def issue_gather(k):
        pltpu.async_copy(h_hbm.at[srcs_v.at[k]], rows_v.at[lax.rem(k, 2)],
                         sem_g)

    def wait_gather(k):
        pltpu.make_async_copy(h_hbm.at[srcs_v.at[k]],
                              rows_v.at[lax.rem(k, 2)], sem_g).wait()

    def issue_scatter(k):
        pltpu.async_copy(rows_v.at[lax.rem(k, 2)], acc_sh.at[dsts_v.at[lax.rem(k, 2)]],
                         sem_s, add=True)

    def wait_scatter(k):
        pltpu.make_async_copy(rows_v.at[lax.rem(k, 2)],
                              acc_sh.at[dsts_v.at[lax.rem(k, 2)]], sem_s).wait()

    issue_dst(0)
    issue_gather(0)

    def edge_body(k, carry):
        @pl.when(k >= 1)
        def _():
            wait_scatter(k - 1)

        @pl.when(k + 1 < NCHUNK)
        def _():
            issue_dst(k + 1)
            issue_gather(k + 1)

        wait_gather(k)
        wait_dst(k)
        issue_scatter(k)
        return carry
    lax.fori_loop(0, NCHUNK, edge_body, 0)
    wait_scatter(NCHUNK - 1)
    plsc.subcore_barrier()

    def out_body(k, carry):
        off = r0 + k * ROW_CHUNK
        pltpu.sync_copy(acc_sh.at[pl.ds(off, ROW_CHUNK)],
                        out_hbm.at[pl.ds(c * N_NODES + off, ROW_CHUNK)])
        return carry
    lax.fori_loop(0, n_row_chunks, out_body, 0)


@jax.jit
def _spmm(h, src3, dst3):
    mesh = plsc.VectorSubcoreMesh(core_axis_name="c", subcore_axis_name="s")
    return pl.kernel(
        _spmm_body,
        out_type=jax.ShapeDtypeStruct((NC * N_NODES, NFEAT), jnp.float32),
        mesh=mesh,
        scratch_types=[
            pltpu.VMEM((NCHUNK, CHUNK), jnp.int32),
            pltpu.VMEM((2, CHUNK), jnp.int32),
            pltpu.VMEM((2, CHUNK, NFEAT), jnp.float32),
            pltpu.SemaphoreType.DMA,
            pltpu.SemaphoreType.DMA,
            pltpu.SemaphoreType.DMA,
            pltpu.VMEM_SHARED((N_ACC, NFEAT), jnp.float32),
        ],
    )(h, src3, dst3)


BR = 1000


def _mlp01_body(x_r, p0_r, p1_r, w0_r, b0_r, w1_r, b1_r, o_r):
    a = p0_r[...] + p1_r[...] - x_r[...]
    h = jnp.dot(a, w0_r[...], preferred_element_type=jnp.float32) + b0_r[...]
    h = jnp.maximum(h, 0.0)
    h = jnp.dot(h, w1_r[...], preferred_element_type=jnp.float32) + b1_r[...]
    o_r[...] = jnp.maximum(h, 0.0)


@jax.jit
def _mlp01(x, p, W0, b0, W1, b1):
    grid = (N_NODES // BR,)
    return pl.pallas_call(
        _mlp01_body,
        grid=grid,
        in_specs=[
            pl.BlockSpec((BR, NFEAT), lambda i: (i, 0)),
            pl.BlockSpec((BR, NFEAT), lambda i: (i, 0)),
            pl.BlockSpec((BR, NFEAT), lambda i: (i + N_NODES // BR, 0)),
            pl.BlockSpec((NFEAT, NFEAT), lambda i: (0, 0)),
            pl.BlockSpec((1, NFEAT), lambda i: (0, 0)),
            pl.BlockSpec((NFEAT, NFEAT), lambda i: (0, 0)),
            pl.BlockSpec((1, NFEAT), lambda i: (0, 0)),
        ],
        out_specs=pl.BlockSpec((BR, NFEAT), lambda i: (i, 0)),
        out_shape=jax.ShapeDtypeStruct((N_NODES, NFEAT), jnp.float32),
    )(x, p, p, W0, b0.reshape(1, NFEAT), W1, b1.reshape(1, NFEAT))


def _mlp23_body(h_r, q0_r, q1_r, w2_r, b2_r, w3_r, b3_r, o_r):
    a = q0_r[...] + q1_r[...] - h_r[...]
    h = jnp.dot(a, w2_r[...], preferred_element_type=jnp.float32) + b2_r[...]
    h = jnp.maximum(h, 0.0)
    logits = jnp.dot(h, w3_r[...], preferred_element_type=jnp.float32) + b3_r[...]
    m = jnp.max(logits, axis=1, keepdims=True)
    z = logits - m
    o_r[...] = z - jnp.log(jnp.sum(jnp.exp(z), axis=1, keepdims=True))


@jax.jit
def _mlp23(h, q, W2, b2, W3p, b3p):
    grid = (N_NODES // BR,)
    return pl.pallas_call(
        _mlp23_body,
        grid=grid,
        in_specs=[
            pl.BlockSpec((BR, NFEAT), lambda i: (i, 0)),
            pl.BlockSpec((BR, NFEAT), lambda i: (i, 0)),
            pl.BlockSpec((BR, NFEAT), lambda i: (i + N_NODES // BR, 0)),
            pl.BlockSpec((NFEAT, NFEAT), lambda i: (0, 0)),
            pl.BlockSpec((1, NFEAT), lambda i: (0, 0)),
            pl.BlockSpec((NFEAT, NFEAT), lambda i: (0, 0)),
            pl.BlockSpec((1, NFEAT), lambda i: (0, 0)),
        ],
        out_specs=pl.BlockSpec((BR, NFEAT), lambda i: (i, 0)),
        out_shape=jax.ShapeDtypeStruct((N_NODES, NFEAT), jnp.float32),
    )(h, q, q, W2, b2.reshape(1, NFEAT), W3p, b3p)


def kernel(x, edge_index, W0, b0, W1, b1, W2, b2, W3, b3):
    dst = edge_index[0]
    src = edge_index[1]
    npad = E_PAD - N_EDGES
    src3 = jnp.concatenate([src, jnp.zeros((npad,), jnp.int32)]
                           ).reshape(NW, NCHUNK, CHUNK)
    dst3 = jnp.concatenate([dst, jnp.full((npad,), N_NODES, jnp.int32)]
                           ).reshape(NW, NCHUNK, CHUNK)

    p = _spmm(x, src3, dst3)
    h2 = _mlp01(x, p, W0, b0, W1, b1)
    q = _spmm(h2, src3, dst3)

    W3p = jnp.zeros((NFEAT, NFEAT), jnp.float32).at[:, :NCLASS].set(W3)
    b3p = jnp.full((1, NFEAT), -1e30, jnp.float32).at[0, :NCLASS].set(b3)
    out = _mlp23(h2, q, W2, b2, W3p, b3p)
    return out[:, :NCLASS]


# speedup vs baseline: 3.0767x; 3.0767x over previous
import functools

import jax
import jax.numpy as jnp
from jax import lax
from jax.experimental import pallas as pl
from jax.experimental.pallas import tpu as pltpu
from jax.experimental.pallas import tpu_sc as plsc

N_NODES = 10000
N_EDGES = 320000
NFEAT = 128
NCLASS = 40

NC = 2
NS = 16
NW = NC * NS

CHUNK = 80
NCHUNK = 125
E_PAD = NW * NCHUNK * CHUNK
N_ACC = N_NODES + 16
ROWS_PER_TILE = 640
ROW_CHUNK = 80


def _spmm_body(h_hbm, src_hbm, dst_hbm, out_hbm,
               srcs_v, dsts_v, rows_v, sem_i, sem_g, sem_s, acc_sh):
    c = lax.axis_index("c")
    s = lax.axis_index("s")
    wid = c * NS + s
    r0 = s * ROWS_PER_TILE
    n_row_chunks = jnp.where(s == NS - 1, 5, 8)

    src_idx = pltpu.async_copy(src_hbm.at[wid], srcs_v, sem_i)

    def init_body(k, carry):
        off = r0 + k * ROW_CHUNK
        pltpu.sync_copy(h_hbm.at[pl.ds(off, ROW_CHUNK)],
                        acc_sh.at[pl.ds(off, ROW_CHUNK)])
        return carry
    lax.fori_loop(0, n_row_chunks, init_body, 0)
    src_idx.wait()
    plsc.subcore_barrier()

    def issue_dst(k):
        pltpu.async_copy(dst_hbm.at[wid, k], dsts_v.at[lax.rem(k, 2)], sem_i)

    def wait_dst(k):
        pltpu.make_async_copy(dst_hbm.at[wid, k],
                              dsts_v.at[lax.rem(k, 2)], sem_i).wait()

    def issue_gather(k):
        pltpu.async_copy(h_hbm.at[srcs_v.at[k]], rows_v.at[lax.rem(k, 2)],
                         sem_g)

    def wait_gather(k):
        pltpu.make_async_copy(h_hbm.at[srcs_v.at[k]],
                              rows_v.at[lax.rem(k, 2)], sem_g).wait()

    def issue_scatter(k):
        pltpu.async_copy(rows_v.at[lax.rem(k, 2)], acc_sh.at[dsts_v.at[lax.rem(k, 2)]],
                         sem_s, add=True)

    def wait_scatter(k):
        pltpu.make_async_copy(rows_v.at[lax.rem(k, 2)],
                              acc_sh.at[dsts_v.at[lax.rem(k, 2)]], sem_s).wait()

    issue_dst(0)
    issue_gather(0)

    def edge_body(k, carry):
        @pl.when(k >= 1)
        def _():
            wait_scatter(k - 1)

        @pl.when(k + 1 < NCHUNK)
        def _():
            issue_dst(k + 1)
            issue_gather(k + 1)

        wait_gather(k)
        wait_dst(k)
        issue_scatter(k)
        return carry
    lax.fori_loop(0, NCHUNK, edge_body, 0)
    wait_scatter(NCHUNK - 1)
    plsc.subcore_barrier()

    def out_body(k, carry):
        off = r0 + k * ROW_CHUNK
        pltpu.sync_copy(acc_sh.at[pl.ds(off, ROW_CHUNK)],
                        out_hbm.at[pl.ds(c * N_NODES + off, ROW_CHUNK)])
        return carry
    lax.fori_loop(0, n_row_chunks, out_body, 0)


@jax.jit
def _spmm(h, src3, dst3):
    mesh = plsc.VectorSubcoreMesh(core_axis_name="c", subcore_axis_name="s")
    return pl.kernel(
        _spmm_body,
        out_type=jax.ShapeDtypeStruct((NC * N_NODES, NFEAT), jnp.float32),
        mesh=mesh,
        scratch_types=[
            pltpu.VMEM((NCHUNK, CHUNK), jnp.int32),
            pltpu.VMEM((2, CHUNK), jnp.int32),
            pltpu.VMEM((2, CHUNK, NFEAT), jnp.float32),
            pltpu.SemaphoreType.DMA,
            pltpu.SemaphoreType.DMA,
            pltpu.SemaphoreType.DMA,
            pltpu.VMEM_SHARED((N_ACC, NFEAT), jnp.float32),
        ],
    )(h, src3, dst3)


BR = 1000


def _mlp01_body(x_r, p0_r, p1_r, w0_r, b0_r, w1_r, b1_r, o_r):
    a = p0_r[...] + p1_r[...] - x_r[...]
    h = jnp.dot(a, w0_r[...], preferred_element_type=jnp.float32) + b0_r[...]
    h = jnp.maximum(h, 0.0)
    h = jnp.dot(h, w1_r[...], preferred_element_type=jnp.float32) + b1_r[...]
    o_r[...] = jnp.maximum(h, 0.0)


@jax.jit
def _mlp01(x, p, W0, b0, W1, b1):
    grid = (N_NODES // BR,)
    return pl.pallas_call(
        _mlp01_body,
        grid=grid,
        in_specs=[
            pl.BlockSpec((BR, NFEAT), lambda i: (i, 0)),
            pl.BlockSpec((BR, NFEAT), lambda i: (i, 0)),
            pl.BlockSpec((BR, NFEAT), lambda i: (i + N_NODES // BR, 0)),
            pl.BlockSpec((NFEAT, NFEAT), lambda i: (0, 0)),
            pl.BlockSpec((1, NFEAT), lambda i: (0, 0)),
            pl.BlockSpec((NFEAT, NFEAT), lambda i: (0, 0)),
            pl.BlockSpec((1, NFEAT), lambda i: (0, 0)),
        ],
        out_specs=pl.BlockSpec((BR, NFEAT), lambda i: (i, 0)),
        out_shape=jax.ShapeDtypeStruct((N_NODES, NFEAT), jnp.float32),
    )(x, p, p, W0, b0.reshape(1, NFEAT), W1, b1.reshape(1, NFEAT))


def _mlp23_body(h_r, q0_r, q1_r, w2_r, b2_r, w3_r, b3_r, o_r):
    a = q0_r[...] + q1_r[...] - h_r[...]
    h = jnp.dot(a, w2_r[...], preferred_element_type=jnp.float32) + b2_r[...]
    h = jnp.maximum(h, 0.0)
    logits = jnp.dot(h, w3_r[...], preferred_element_type=jnp.float32) + b3_r[...]
    m = jnp.max(logits, axis=1, keepdims=True)
    z = logits - m
    o_r[...] = z - jnp.log(jnp.sum(jnp.exp(z), axis=1, keepdims=True))


@jax.jit
def _mlp23(h, q, W2, b2, W3p, b3p):
    grid = (N_NODES // BR,)
    return pl.pallas_call(
        _mlp23_body,
        grid=grid,
        in_specs=[
            pl.BlockSpec((BR, NFEAT), lambda i: (i, 0)),
            pl.BlockSpec((BR, NFEAT), lambda i: (i, 0)),
            pl.BlockSpec((BR, NFEAT), lambda i: (i + N_NODES // BR, 0)),
            pl.BlockSpec((NFEAT, NFEAT), lambda i: (0, 0)),
            pl.BlockSpec((1, NFEAT), lambda i: (0, 0)),
            pl.BlockSpec((NFEAT, NFEAT), lambda i: (0, 0)),
            pl.BlockSpec((1, NFEAT), lambda i: (0, 0)),
        ],
        out_specs=pl.BlockSpec((BR, NFEAT), lambda i: (i, 0)),
        out_shape=jax.ShapeDtypeStruct((N_NODES, NFEAT), jnp.float32),
    )(h, q, q, W2, b2.reshape(1, NFEAT), W3p, b3p)


def kernel(x, edge_index, W0, b0, W1, b1, W2, b2, W3, b3):
    dst = edge_index[0]
    src = edge_index[1]
    npad = E_PAD - N_EDGES
    src3 = jnp.concatenate([src, jnp.zeros((npad,), jnp.int32)]
                           ).reshape(NW, NCHUNK, CHUNK)
    dst3 = jnp.concatenate([dst, jnp.full((npad,), N_NODES, jnp.int32)]
                           ).reshape(NW, NCHUNK, CHUNK)

    p = _spmm(x, src3, dst3)
    h2 = _mlp01(x, p, W0, b0, W1, b1)
    q = _spmm(h2, src3, dst3)

    W3p = jnp.zeros((NFEAT, NFEAT), jnp.float32).at[:, :NCLASS].set(W3)
    b3p = jnp.full((1, NFEAT), -1e30, jnp.float32).at[0, :NCLASS].set(b3)
    out = _mlp23(h2, q, W2, b2, W3p, b3p)
    return out[:, :NCLASS]
